# baseline (speedup 1.0000x reference)
import jax
import jax.numpy as jnp
from jax.experimental import pallas as pl

N = 100000


def _bias_kernel(h_ref, b_ref, o_ref):
    o_ref[...] = h_ref[...] + b_ref[0]


def kernel(x, edge_index, W1, b1, W2, b2, W3, b3, W4, b4):
    src0 = edge_index[0]
    dst0 = edge_index[1]
    loop = jnp.arange(N, dtype=src0.dtype)
    src = jnp.concatenate([src0, loop])
    dst = jnp.concatenate([dst0, loop])
    deg = jnp.zeros((N,), jnp.float32).at[dst].add(1.0)
    dinv = jnp.where(deg > 0, jax.lax.rsqrt(jnp.maximum(deg, 1e-12)), 0.0)
    norm = dinv[src] * dinv[dst]

    def gcn_conv(h, W, b):
        h = h @ W
        msg = h[src] * norm[:, None]
        out = jnp.zeros((N, W.shape[1]), jnp.float32).at[dst].add(msg)
        return out + b

    h = jax.nn.leaky_relu(gcn_conv(x, W1, b1), negative_slope=0.01)
    h = jax.nn.leaky_relu(gcn_conv(h, W2, b2), negative_slope=0.01)
    h = jax.nn.leaky_relu(gcn_conv(h, W3, b3), negative_slope=0.01)
    h = gcn_conv(h, W4, jnp.zeros_like(b4))
    out = pl.pallas_call(
        _bias_kernel,
        grid=(100,),
        in_specs=[pl.BlockSpec((1000, 1), lambda i: (i, 0)),
                  pl.BlockSpec((1,), lambda i: (0,))],
        out_specs=pl.BlockSpec((1000, 1), lambda i: (i, 0)),
        out_shape=jax.ShapeDtypeStruct((N, 1), jnp.float32),
    )(h, b4)
    return out


# speedup vs baseline: 14.4595x; 14.4595x over previous
import functools

import jax
import jax.numpy as jnp
from jax import lax
from jax.experimental import pallas as pl
from jax.experimental.pallas import tpu as pltpu
from jax.experimental.pallas import tpu_sc as plsc

N = 100000
E = 1600000
NC = 2
NS = 16
NW = NC * NS
BLK = 1024
IDXR = 8
BPW = 49
E_PAD = NW * BLK * BPW
NOUT = 100352
NACC = NOUT
RPT = NOUT // NS
BUFR = 392
BN = 1000
GRID = N // BN


def _edge_loop(with_gather, src2, dst2, table, src_v, dst_v, msg_v, acc,
               semg, sems):
    wid = lax.axis_index("c") * NS + lax.axis_index("s")
    base_row = wid * (BPW * IDXR)

    def blk(b, carry):
        r = base_row + b * IDXR
        pltpu.sync_copy(dst2.at[pl.ds(r, IDXR)], dst_v)
        if with_gather:
            pltpu.sync_copy(src2.at[pl.ds(r, IDXR)], src_v)
            gs = [
                pltpu.async_copy(table.at[src_v.at[j]],
                                 msg_v.at[pl.ds(j * 128, 128)], semg)
                for j in range(IDXR)
            ]
            for h in gs:
                h.wait()
        ss = [
            pltpu.async_copy(
                msg_v.at[pl.ds((j * 128) if with_gather else 0, 128)],
                acc.at[dst_v.at[j]], sems, add=True)
            for j in range(IDXR)
        ]
        for h in ss:
            h.wait()
        return carry

    lax.fori_loop(0, BPW, blk, 0)


def _zero_acc(buf_v, acc):
    s = lax.axis_index("s")

    def zrow(i, c):
        buf_v[i] = jnp.zeros((16,), jnp.float32)
        return c

    lax.fori_loop(0, BUFR, zrow, 0)
    for k in range(RPT // BUFR):
        pltpu.sync_copy(buf_v, acc.at[pl.ds(s * RPT + k * BUFR, BUFR)])


def _copy_out(out2, buf_v, acc):
    s = lax.axis_index("s")
    c = lax.axis_index("c")
    for k in range(RPT // BUFR):
        rb = s * RPT + k * BUFR
        pltpu.sync_copy(acc.at[pl.ds(rb, BUFR)], buf_v)
        pltpu.sync_copy(buf_v, out2.at[pl.ds(c * NOUT + rb, BUFR)])


def _agg_body(src2, dst2, table, out2, src_v, dst_v, msg_v, buf_v,
              acc, semg, sems):
    _zero_acc(buf_v, acc)
    plsc.subcore_barrier()
    _edge_loop(True, src2, dst2, table, src_v, dst_v, msg_v, acc, semg, sems)
    plsc.subcore_barrier()
    _copy_out(out2, buf_v, acc)


def _deg_body(dst2, out2, dst_v, ones_v, buf_v, acc, sems):
    def orow(i, c):
        ones_v[i] = jnp.ones((16,), jnp.float32)
        return c

    lax.fori_loop(0, 128, orow, 0)
    _zero_acc(buf_v, acc)
    plsc.subcore_barrier()
    _edge_loop(False, None, dst2, None, None, dst_v, ones_v, acc, None, sems)
    plsc.subcore_barrier()
    _copy_out(out2, buf_v, acc)


def _sc_mesh():
    return plsc.VectorSubcoreMesh(core_axis_name="c", subcore_axis_name="s")


_SC_PARAMS = pltpu.CompilerParams(use_tc_tiling_on_sc=False)

_agg16 = pl.kernel(
    _agg_body,
    out_type=jax.ShapeDtypeStruct((2 * NOUT, 16), jnp.float32),
    mesh=_sc_mesh(),
    compiler_params=_SC_PARAMS,
    scratch_types=[
        pltpu.VMEM((IDXR, 128), jnp.int32),
        pltpu.VMEM((IDXR, 128), jnp.int32),
        pltpu.VMEM((BLK, 16), jnp.float32),
        pltpu.VMEM((BUFR, 16), jnp.float32),
        pltpu.VMEM_SHARED((NACC, 16), jnp.float32),
        pltpu.SemaphoreType.DMA,
        pltpu.SemaphoreType.DMA,
    ],
)

_deg16 = pl.kernel(
    _deg_body,
    out_type=jax.ShapeDtypeStruct((2 * NOUT, 16), jnp.float32),
    mesh=_sc_mesh(),
    compiler_params=_SC_PARAMS,
    scratch_types=[
        pltpu.VMEM((IDXR, 128), jnp.int32),
        pltpu.VMEM((128, 16), jnp.float32),
        pltpu.VMEM((BUFR, 16), jnp.float32),
        pltpu.VMEM_SHARED((NACC, 16), jnp.float32),
        pltpu.SemaphoreType.DMA,
    ],
)


def _lrelu(h):
    return jnp.where(h >= 0.0, h, 0.01 * h)


def _stage_a(x_ref, da_ref, db_ref, dinv_ref, t0_ref):
    deg = 1.0 + da_ref[:, 0:1] + db_ref[:, 0:1]
    dinv = lax.rsqrt(deg)
    dinv_ref[...] = dinv
    xs = x_ref[...] * dinv
    t0_ref[...] = jnp.concatenate(
        [xs, jnp.zeros((BN, 10), jnp.float32)], axis=1)


def _stage_b(s1a_ref, s1b_ref, t0_ref, dinv_ref, w1_ref, b1_ref, t1_ref):
    dinv = dinv_ref[...]
    p1 = ((s1a_ref[...] + s1b_ref[...] + t0_ref[...]) * dinv)[:, :6]
    h1 = _lrelu(jnp.dot(p1, w1_ref[...],
                        preferred_element_type=jnp.float32) + b1_ref[...])
    t1_ref[...] = h1 * dinv


def _stage_c(sa0, sb0, sa1, sb1, sa2, sb2, sa3, sb3, t1_ref, dinv_ref,
             w2_ref, b2_ref, w3_ref, t3_ref):
    dinv = dinv_ref[...]
    s2 = jnp.concatenate(
        [sa0[...] + sb0[...], sa1[...] + sb1[...],
         sa2[...] + sb2[...], sa3[...] + sb3[...]], axis=1)
    p2 = (s2 + t1_ref[...]) * dinv
    h2 = _lrelu(jnp.dot(p2, w2_ref[...],
                        preferred_element_type=jnp.float32) + b2_ref[...])
    m3 = jnp.dot(h2, w3_ref[...], preferred_element_type=jnp.float32)
    t3_ref[...] = m3 * dinv


def _stage_d(sa0, sb0, sa1, sb1, sa2, sb2, sa3, sb3, t3_ref, dinv_ref,
             b3_ref, w4_ref, t4_ref):
    dinv = dinv_ref[...]
    s3 = jnp.concatenate(
        [sa0[...] + sb0[...], sa1[...] + sb1[...],
         sa2[...] + sb2[...], sa3[...] + sb3[...]], axis=1)
    h3 = _lrelu((s3 + t3_ref[...]) * dinv + b3_ref[...])
    m4 = jnp.dot(h3, w4_ref[...], preferred_element_type=jnp.float32)
    t4_ref[...] = jnp.concatenate(
        [m4 * dinv, jnp.zeros((BN, 15), jnp.float32)], axis=1)


def _stage_e(s4a_ref, s4b_ref, t4_ref, dinv_ref, b4_ref, out_ref):
    p4 = (s4a_ref[...] + s4b_ref[...] + t4_ref[...]) * dinv_ref[...]
    out_ref[...] = p4[:, :1] + b4_ref[...]


def _row_spec(w):
    return pl.BlockSpec((BN, w), lambda i: (i, 0))


def _full_spec(shape):
    return pl.BlockSpec(shape, lambda i: tuple(0 for _ in shape))


def _tc_call(body, in_widths, const_shapes, out_widths):
    in_specs = ([_row_spec(w) for w in in_widths]
                + [_full_spec(s) for s in const_shapes])
    if len(out_widths) == 1:
        out_specs = _row_spec(out_widths[0])
        out_shape = jax.ShapeDtypeStruct((N, out_widths[0]), jnp.float32)
    else:
        out_specs = [_row_spec(w) for w in out_widths]
        out_shape = [jax.ShapeDtypeStruct((N, w), jnp.float32)
                     for w in out_widths]
    return pl.pallas_call(
        body, grid=(GRID,), in_specs=in_specs, out_specs=out_specs,
        out_shape=out_shape)


def _halves(s):
    return s[:N], s[NOUT:NOUT + N]


def kernel(x, edge_index, W1, b1, W2, b2, W3, b3, W4, b4):
    src = edge_index[0]
    dst = edge_index[1]
    pad = E_PAD - E
    srcp = jnp.concatenate([src, jnp.zeros((pad,), jnp.int32)])
    dstp = jnp.concatenate([dst, jnp.full((pad,), N, jnp.int32)])
    src2 = srcp.reshape(E_PAD // 128, 128)
    dst2 = dstp.reshape(E_PAD // 128, 128)

    b1r = b1.reshape(1, 64)
    b2r = b2.reshape(1, 128)
    b3r = b3.reshape(1, 64)
    b4r = b4.reshape(1, 1)

    dga, dgb = _halves(_deg16(dst2))

    dinv, t0 = _tc_call(_stage_a, [6, 16, 16], [], [1, 16])(x, dga, dgb)

    s1a, s1b = _halves(_agg16(src2, dst2, t0))
    t1 = _tc_call(_stage_b, [16, 16, 16, 1], [(6, 64), (1, 64)], [64])(
        s1a, s1b, t0, dinv, W1, b1r)

    s2 = [_halves(_agg16(src2, dst2,
                         lax.slice(t1, (0, 16 * c), (N, 16 * c + 16))))
          for c in range(4)]
    s2flat = [h for pair in s2 for h in pair]
    t3 = _tc_call(_stage_c, [16] * 8 + [64, 1],
                  [(64, 128), (1, 128), (128, 64)], [64])(
        *s2flat, t1, dinv, W2, b2r, W3)

    s3 = [_halves(_agg16(src2, dst2,
                         lax.slice(t3, (0, 16 * c), (N, 16 * c + 16))))
          for c in range(4)]
    s3flat = [h for pair in s3 for h in pair]
    t4 = _tc_call(_stage_d, [16] * 8 + [64, 1], [(1, 64), (64, 1)], [16])(
        *s3flat, t3, dinv, b3r, W4)

    s4a, s4b = _halves(_agg16(src2, dst2, t4))
    out = _tc_call(_stage_e, [16, 16, 16, 1], [(1, 1)], [1])(
        s4a, s4b, t4, dinv, b4r)
    return out


# speedup vs baseline: 17.2199x; 1.1909x over previous
import functools

import jax
import jax.numpy as jnp
from jax import lax
from jax.experimental import pallas as pl
from jax.experimental.pallas import tpu as pltpu
from jax.experimental.pallas import tpu_sc as plsc

N = 100000
E = 1600000
NC = 2
NS = 16
NW = NC * NS
SSR = 4
SSE = SSR * 128
E_PAD = 1605632
NSS = E_PAD // SSE
SS_W = NSS // NW
SS_T = NSS // NS
NOUT = 100352
NACC = NOUT
RPT = NOUT // NS
BUFR = 392
BN = 1000
GRID = N // BN


def _add_off(sv, off16):
    for r in range(SSR):
        for l in range(0, 128, 16):
            sv[r, pl.ds(l, 16)] = sv[r, pl.ds(l, 16)] + off16


def _run_edges(src2, dst2, table, acc, sv, dv, mv, semi, semg, sems,
               base_ss, n_ss, off):
    with_gather = table is not None
    off16 = None if off is None else jnp.full((16,), off, jnp.int32)

    def pair(g, carry):
        hs = []
        for i in (0, 1):
            r = (base_ss + 2 * g + i) * SSR
            if with_gather:
                hs.append(pltpu.async_copy(src2.at[pl.ds(r, SSR)], sv[i],
                                           semi))
            hs.append(pltpu.async_copy(dst2.at[pl.ds(r, SSR)], dv[i], semi))
        for h in hs:
            h.wait()
        if with_gather:
            if off16 is not None:
                _add_off(sv[0], off16)
                _add_off(sv[1], off16)
            g0 = [pltpu.async_copy(table.at[sv[0].at[j]], mv[0].at[j], semg)
                  for j in range(SSR)]
            g1 = [pltpu.async_copy(table.at[sv[1].at[j]], mv[1].at[j], semg)
                  for j in range(SSR)]
            for h in g0:
                h.wait()
            s0 = [pltpu.async_copy(mv[0].at[j], acc.at[dv[0].at[j]], sems,
                                   add=True) for j in range(SSR)]
            for h in g1:
                h.wait()
            s1 = [pltpu.async_copy(mv[1].at[j], acc.at[dv[1].at[j]], sems,
                                   add=True) for j in range(SSR)]
        else:
            s0 = [pltpu.async_copy(mv[0], acc.at[dv[0].at[j]], sems,
                                   add=True) for j in range(SSR)]
            s1 = [pltpu.async_copy(mv[1], acc.at[dv[1].at[j]], sems,
                                   add=True) for j in range(SSR)]
        for h in s0:
            h.wait()
        for h in s1:
            h.wait()
        return carry

    lax.fori_loop(0, n_ss // 2, pair, 0)


def _zero_acc(buf_v, acc):
    s = lax.axis_index("s")

    def zrow(i, c):
        buf_v[i] = jnp.zeros((16,), jnp.float32)
        return c

    lax.fori_loop(0, BUFR, zrow, 0)
    for k in range(RPT // BUFR):
        pltpu.sync_copy(buf_v, acc.at[pl.ds(s * RPT + k * BUFR, BUFR)])


def _copy_out(out2, buf_v, acc, out_base):
    s = lax.axis_index("s")
    for k in range(RPT // BUFR):
        rb = s * RPT + k * BUFR
        pltpu.sync_copy(acc.at[pl.ds(rb, BUFR)], buf_v)
        pltpu.sync_copy(buf_v, out2.at[pl.ds(out_base + rb, BUFR)])


def _agg_body(src2, dst2, table, out2, sv0, sv1, dv0, dv1, mv0, mv1, buf_v,
              acc, semi, semg, sems):
    c = lax.axis_index("c")
    s = lax.axis_index("s")
    wid = c * NS + s
    _zero_acc(buf_v, acc)
    plsc.subcore_barrier()
    _run_edges(src2, dst2, table, acc, [sv0, sv1], [dv0, dv1], [mv0, mv1],
               semi, semg, sems, wid * SS_W, SS_W, None)
    plsc.subcore_barrier()
    _copy_out(out2, buf_v, acc, c * NOUT)


def _agg4_body(src2, dst2, table, out2, sv0, sv1, dv0, dv1, mv0, mv1, buf_v,
               acc, semi, semg, sems):
    c = lax.axis_index("c")
    s = lax.axis_index("s")
    for cc in range(2):
        chunk = c * 2 + cc
        _zero_acc(buf_v, acc)
        plsc.subcore_barrier()
        _run_edges(src2, dst2, table, acc, [sv0, sv1], [dv0, dv1],
                   [mv0, mv1], semi, semg, sems, s * SS_T, SS_T, chunk * N)
        plsc.subcore_barrier()
        _copy_out(out2, buf_v, acc, chunk * NOUT)
        if cc == 0:
            plsc.subcore_barrier()


def _deg_body(dst2, out2, dv0, dv1, ones_v, buf_v, acc, semi, sems):
    def orow(i, carry):
        ones_v[i] = jnp.ones((16,), jnp.float32)
        return carry

    lax.fori_loop(0, 128, orow, 0)
    c = lax.axis_index("c")
    s = lax.axis_index("s")
    wid = c * NS + s
    _zero_acc(buf_v, acc)
    plsc.subcore_barrier()
    _run_edges(None, dst2, None, acc, None, [dv0, dv1], [ones_v, ones_v],
               semi, None, sems, wid * SS_W, SS_W, None)
    plsc.subcore_barrier()
    _copy_out(out2, buf_v, acc, c * NOUT)


def _sc_mesh():
    return plsc.VectorSubcoreMesh(core_axis_name="c", subcore_axis_name="s")


_SC_PARAMS = pltpu.CompilerParams(use_tc_tiling_on_sc=False)

_AGG_SCRATCH = [
    pltpu.VMEM((SSR, 128), jnp.int32),
    pltpu.VMEM((SSR, 128), jnp.int32),
    pltpu.VMEM((SSR, 128), jnp.int32),
    pltpu.VMEM((SSR, 128), jnp.int32),
    pltpu.VMEM((SSR, 128, 16), jnp.float32),
    pltpu.VMEM((SSR, 128, 16), jnp.float32),
    pltpu.VMEM((BUFR, 16), jnp.float32),
    pltpu.VMEM_SHARED((NACC, 16), jnp.float32),
    pltpu.SemaphoreType.DMA,
    pltpu.SemaphoreType.DMA,
    pltpu.SemaphoreType.DMA,
]

_agg16 = pl.kernel(
    _agg_body,
    out_type=jax.ShapeDtypeStruct((2 * NOUT, 16), jnp.float32),
    mesh=_sc_mesh(),
    compiler_params=_SC_PARAMS,
    scratch_types=_AGG_SCRATCH,
)

_agg4x16 = pl.kernel(
    _agg4_body,
    out_type=jax.ShapeDtypeStruct((4 * NOUT, 16), jnp.float32),
    mesh=_sc_mesh(),
    compiler_params=_SC_PARAMS,
    scratch_types=_AGG_SCRATCH,
)

_deg16 = pl.kernel(
    _deg_body,
    out_type=jax.ShapeDtypeStruct((2 * NOUT, 16), jnp.float32),
    mesh=_sc_mesh(),
    compiler_params=_SC_PARAMS,
    scratch_types=[
        pltpu.VMEM((SSR, 128), jnp.int32),
        pltpu.VMEM((SSR, 128), jnp.int32),
        pltpu.VMEM((128, 16), jnp.float32),
        pltpu.VMEM((BUFR, 16), jnp.float32),
        pltpu.VMEM_SHARED((NACC, 16), jnp.float32),
        pltpu.SemaphoreType.DMA,
        pltpu.SemaphoreType.DMA,
    ],
)


def _lrelu(h):
    return jnp.where(h >= 0.0, h, 0.01 * h)


def _stage_a(x_ref, da_ref, db_ref, dinv_ref, t0_ref):
    deg = 1.0 + da_ref[:, 0:1] + db_ref[:, 0:1]
    dinv = lax.rsqrt(deg)
    dinv_ref[...] = dinv
    xs = x_ref[...] * dinv
    t0_ref[...] = jnp.concatenate(
        [xs, jnp.zeros((BN, 10), jnp.float32)], axis=1)


def _stage_b(s1a_ref, s1b_ref, t0_ref, dinv_ref, w1_ref, b1_ref, t1_ref):
    dinv = dinv_ref[...]
    p1 = ((s1a_ref[...] + s1b_ref[...] + t0_ref[...]) * dinv)[:, :6]
    h1 = _lrelu(jnp.dot(p1, w1_ref[...],
                        preferred_element_type=jnp.float32) + b1_ref[...])
    t1_ref[...] = h1 * dinv


def _stage_c(sc0, sc1, sc2, sc3, t1_ref, dinv_ref,
             w2_ref, b2_ref, w3_ref, t3_ref):
    dinv = dinv_ref[...]
    s2 = jnp.concatenate([sc0[...], sc1[...], sc2[...], sc3[...]], axis=1)
    p2 = (s2 + t1_ref[...]) * dinv
    h2 = _lrelu(jnp.dot(p2, w2_ref[...],
                        preferred_element_type=jnp.float32) + b2_ref[...])
    m3 = jnp.dot(h2, w3_ref[...], preferred_element_type=jnp.float32)
    t3_ref[...] = m3 * dinv


def _stage_d(sc0, sc1, sc2, sc3, t3_ref, dinv_ref,
             b3_ref, w4_ref, t4_ref):
    dinv = dinv_ref[...]
    s3 = jnp.concatenate([sc0[...], sc1[...], sc2[...], sc3[...]], axis=1)
    h3 = _lrelu((s3 + t3_ref[...]) * dinv + b3_ref[...])
    m4 = jnp.dot(h3, w4_ref[...], preferred_element_type=jnp.float32)
    t4_ref[...] = jnp.concatenate(
        [m4 * dinv, jnp.zeros((BN, 15), jnp.float32)], axis=1)


def _stage_e(s4a_ref, s4b_ref, t4_ref, dinv_ref, b4_ref, out_ref):
    p4 = (s4a_ref[...] + s4b_ref[...] + t4_ref[...]) * dinv_ref[...]
    out_ref[...] = p4[:, :1] + b4_ref[...]


def _row_spec(w):
    return pl.BlockSpec((BN, w), lambda i: (i, 0))


def _full_spec(shape):
    return pl.BlockSpec(shape, lambda i: tuple(0 for _ in shape))


def _tc_call(body, in_widths, const_shapes, out_widths):
    in_specs = ([_row_spec(w) for w in in_widths]
                + [_full_spec(s) for s in const_shapes])
    if len(out_widths) == 1:
        out_specs = _row_spec(out_widths[0])
        out_shape = jax.ShapeDtypeStruct((N, out_widths[0]), jnp.float32)
    else:
        out_specs = [_row_spec(w) for w in out_widths]
        out_shape = [jax.ShapeDtypeStruct((N, w), jnp.float32)
                     for w in out_widths]
    return pl.pallas_call(
        body, grid=(GRID,), in_specs=in_specs, out_specs=out_specs,
        out_shape=out_shape)


def _halves(s):
    return s[:N], s[NOUT:NOUT + N]


def kernel(x, edge_index, W1, b1, W2, b2, W3, b3, W4, b4):
    src = edge_index[0]
    dst = edge_index[1]
    pad = E_PAD - E
    srcp = jnp.concatenate([src, jnp.zeros((pad,), jnp.int32)])
    dstp = jnp.concatenate([dst, jnp.full((pad,), N, jnp.int32)])
    src2 = srcp.reshape(E_PAD // 128, 128)
    dst2 = dstp.reshape(E_PAD // 128, 128)

    b1r = b1.reshape(1, 64)
    b2r = b2.reshape(1, 128)
    b3r = b3.reshape(1, 64)
    b4r = b4.reshape(1, 1)

    dga, dgb = _halves(_deg16(dst2))

    dinv, t0 = _tc_call(_stage_a, [6, 16, 16], [], [1, 16])(x, dga, dgb)

    s1a, s1b = _halves(_agg16(src2, dst2, t0))
    t1 = _tc_call(_stage_b, [16, 16, 16, 1], [(6, 64), (1, 64)], [64])(
        s1a, s1b, t0, dinv, W1, b1r)

    t1s = t1.reshape(N, 4, 16).transpose(1, 0, 2).reshape(4 * N, 16)
    s2full = _agg4x16(src2, dst2, t1s)
    s2c = [lax.slice(s2full, (c * NOUT, 0), (c * NOUT + N, 16))
           for c in range(4)]
    t3 = _tc_call(_stage_c, [16] * 4 + [64, 1],
                  [(64, 128), (1, 128), (128, 64)], [64])(
        *s2c, t1, dinv, W2, b2r, W3)

    t3s = t3.reshape(N, 4, 16).transpose(1, 0, 2).reshape(4 * N, 16)
    s3full = _agg4x16(src2, dst2, t3s)
    s3c = [lax.slice(s3full, (c * NOUT, 0), (c * NOUT + N, 16))
           for c in range(4)]
    t4 = _tc_call(_stage_d, [16] * 4 + [64, 1], [(1, 64), (64, 1)], [16])(
        *s3c, t3, dinv, b3r, W4)

    s4a, s4b = _halves(_agg16(src2, dst2, t4))
    out = _tc_call(_stage_e, [16, 16, 16, 1], [(1, 1)], [1])(
        s4a, s4b, t4, dinv, b4r)
    return out
